# baseline (speedup 1.0000x reference)
import functools

import jax
import jax.numpy as jnp
from jax import lax
from jax.experimental import pallas as pl
from jax.experimental.pallas import tpu as pltpu
from jax.experimental.pallas import tpu_sc as plsc

V = 1000000
D = 32
B = 16384
L = 200
NC = 2
NS = 16
NW = NC * NS
UNIT = 512
N_UNITS = B * L // UNIT
PER_W = N_UNITS // NW
ROUNDS = PER_W // 2


def _sc_kernel():
    mesh = plsc.VectorSubcoreMesh(core_axis_name="c", subcore_axis_name="s")

    scratch = (
        [pltpu.VMEM((UNIT,), jnp.int32) for _ in range(2)]
        + [pltpu.VMEM((UNIT, D), jnp.float32) for _ in range(2)]
        + [pltpu.VMEM((4, 4, 8, 128), jnp.float32) for _ in range(2)]
        + [pltpu.SemaphoreType.DMA for _ in range(4)]
    )

    @functools.partial(
        pl.kernel,
        mesh=mesh,
        out_type=jax.ShapeDtypeStruct((L, D // 8, B // 128, 8, 128), jnp.float32),
        scratch_types=scratch,
        compiler_params=pltpu.CompilerParams(
            use_tc_tiling_on_sc=False, needs_layout_passes=False),
    )
    def sc_gather(q_hbm, table_hbm, out_hbm, idx0, idx1, r0, r1, t0, t1,
                  gsem0, gsem1, wsem0, wsem1):
        idx_v = (idx0, idx1)
        r_v = (r0, r1)
        t_v = (t0, t1)
        gsem = (gsem0, gsem1)
        wsem = (wsem0, wsem1)

        wid = lax.axis_index("s") * NC + lax.axis_index("c")
        u_base = wid * PER_W
        ramp = lax.iota(jnp.int32, 16)

        def idx_copy(p, u):
            return pltpu.make_async_copy(
                q_hbm.at[pl.ds(u * UNIT, UNIT)], idx_v[p], gsem[p])

        def gather_copy(p):
            return pltpu.make_async_copy(table_hbm.at[idx_v[p]], r_v[p], gsem[p])

        def out_copy(p, u):
            rt = u // 256
            rem = u % 256
            ct = rem // 2
            h = rem % 2
            l0 = rt * 8 + h * 4
            return pltpu.make_async_copy(
                t_v[p], out_hbm.at[pl.ds(l0, 4), :, ct], wsem[p])

        def transpose(p):
            def trow(row, carry):
                lrh = row // 32
                d0 = row % 32
                tr = d0 // 8
                dr = d0 % 8
                cols = jnp.broadcast_to(d0, (16,))
                l_s = jnp.broadcast_to(lrh, (16,))
                tr_s = jnp.broadcast_to(tr, (16,))
                dr_s = jnp.broadcast_to(dr, (16,))
                for k in range(8):
                    rows = ramp + (lrh * 128 + k * 16)
                    vals = plsc.load_gather(r_v[p], [rows, cols])
                    plsc.store_scatter(t_v[p], [l_s, tr_s, dr_s, ramp + k * 16], vals)
                return carry

            lax.fori_loop(0, 128, trow, 0)

        idx_copy(0, u_base).start()
        idx_copy(0, u_base).wait()
        gather_copy(0).start()
        idx_copy(1, u_base + 1).start()
        idx_copy(1, u_base + 1).wait()
        gather_copy(1).start()

        def body(rnd, carry):
            ua = u_base + 2 * rnd
            ub = ua + 1
            gather_copy(0).wait()

            @pl.when(rnd > 0)
            def _():
                out_copy(0, ua - 2).wait()

            transpose(0)
            out_copy(0, ua).start()

            @pl.when(rnd + 1 < ROUNDS)
            def _():
                idx_copy(0, ua + 2).start()
                idx_copy(0, ua + 2).wait()
                gather_copy(0).start()

            gather_copy(1).wait()

            @pl.when(rnd > 0)
            def _():
                out_copy(1, ub - 2).wait()

            transpose(1)
            out_copy(1, ub).start()

            @pl.when(rnd + 1 < ROUNDS)
            def _():
                idx_copy(1, ub + 2).start()
                idx_copy(1, ub + 2).wait()
                gather_copy(1).start()

            return carry

        lax.fori_loop(0, ROUNDS, body, 0)

        last_a = u_base + 2 * (ROUNDS - 1)
        out_copy(0, last_a).wait()
        out_copy(1, last_a + 1).wait()

    return sc_gather


def kernel(sentence, table):
    q = sentence.astype(jnp.int32).reshape(128, 128, 25, 8).transpose(2, 0, 3, 1)
    q_flat = q.reshape(B * L)
    out5 = _sc_kernel()(q_flat, table)
    return out5.transpose(2, 4, 0, 1, 3).reshape(B, L, D)


# speedup vs baseline: 2.0569x; 2.0569x over previous
import functools

import jax
import jax.numpy as jnp
from jax import lax
from jax.experimental import pallas as pl
from jax.experimental.pallas import tpu as pltpu
from jax.experimental.pallas import tpu_sc as plsc

V = 1000000
D = 32
B = 16384
L = 200
NC = 2
NS = 16
NW = NC * NS
UNIT = 512
N_UNITS = B * L // UNIT
PER_W = N_UNITS // NW
ROUNDS = PER_W // 2


def _sc_kernel():
    mesh = plsc.VectorSubcoreMesh(core_axis_name="c", subcore_axis_name="s")

    scratch = (
        [pltpu.VMEM((UNIT,), jnp.int32) for _ in range(2)]
        + [pltpu.VMEM((UNIT, D), jnp.float32) for _ in range(2)]
        + [pltpu.VMEM((4, 4, 8, 129), jnp.float32) for _ in range(2)]
        + [pltpu.SemaphoreType.DMA for _ in range(4)]
    )

    @functools.partial(
        pl.kernel,
        mesh=mesh,
        out_type=jax.ShapeDtypeStruct((L, D // 8, B // 128, 8, 128), jnp.float32),
        scratch_types=scratch,
        compiler_params=pltpu.CompilerParams(
            use_tc_tiling_on_sc=False, needs_layout_passes=False),
    )
    def sc_gather(q_hbm, table_hbm, out_hbm, idx0, idx1, r0, r1, t0, t1,
                  gsem0, gsem1, wsem0, wsem1):
        idx_v = (idx0, idx1)
        r_v = (r0, r1)
        t_v = (t0, t1)
        gsem = (gsem0, gsem1)
        wsem = (wsem0, wsem1)

        wid = lax.axis_index("s") * NC + lax.axis_index("c")
        u_base = wid * PER_W
        ramp = lax.iota(jnp.int32, 16)

        def idx_copy(p, u):
            return pltpu.make_async_copy(
                q_hbm.at[pl.ds(u * UNIT, UNIT)], idx_v[p], gsem[p])

        def gather_copy(p):
            return pltpu.make_async_copy(table_hbm.at[idx_v[p]], r_v[p], gsem[p])

        def out_copy(p, u):
            rt = u // 256
            rem = u % 256
            ct = rem // 2
            h = rem % 2
            l0 = rt * 8 + h * 4
            return pltpu.make_async_copy(
                t_v[p].at[:, :, :, pl.ds(0, 128)],
                out_hbm.at[pl.ds(l0, 4), :, ct], wsem[p])

        tr_vec0 = ramp // 8
        dr_vec0 = lax.rem(ramp, 8)
        tr_vec1 = (ramp + 16) // 8
        dr_vec1 = lax.rem(ramp + 16, 8)

        def transpose(p):
            def tl(lrh, carry):
                l_s = jnp.broadcast_to(lrh, (16,))

                def tb(bcj, carry2):
                    for m in range(4):
                        bc = bcj * 4 + m
                        row = lrh * 128 + bc
                        bc_s = jnp.broadcast_to(bc, (16,))
                        v0 = r_v[p][row, pl.ds(0, 16)]
                        plsc.store_scatter(
                            t_v[p], [l_s, tr_vec0, dr_vec0, bc_s], v0)
                        v1 = r_v[p][row, pl.ds(16, 16)]
                        plsc.store_scatter(
                            t_v[p], [l_s, tr_vec1, dr_vec1, bc_s], v1)
                    return carry2

                lax.fori_loop(0, 32, tb, 0)
                return carry

            lax.fori_loop(0, 4, tl, 0)

        idx_copy(0, u_base).start()
        idx_copy(0, u_base).wait()
        gather_copy(0).start()
        idx_copy(1, u_base + 1).start()
        idx_copy(1, u_base + 1).wait()
        gather_copy(1).start()

        def body(rnd, carry):
            ua = u_base + 2 * rnd
            ub = ua + 1
            gather_copy(0).wait()

            @pl.when(rnd > 0)
            def _():
                out_copy(0, ua - 2).wait()

            transpose(0)
            out_copy(0, ua).start()

            @pl.when(rnd + 1 < ROUNDS)
            def _():
                idx_copy(0, ua + 2).start()
                idx_copy(0, ua + 2).wait()
                gather_copy(0).start()

            gather_copy(1).wait()

            @pl.when(rnd > 0)
            def _():
                out_copy(1, ub - 2).wait()

            transpose(1)
            out_copy(1, ub).start()

            @pl.when(rnd + 1 < ROUNDS)
            def _():
                idx_copy(1, ub + 2).start()
                idx_copy(1, ub + 2).wait()
                gather_copy(1).start()

            return carry

        lax.fori_loop(0, ROUNDS, body, 0)

        last_a = u_base + 2 * (ROUNDS - 1)
        out_copy(0, last_a).wait()
        out_copy(1, last_a + 1).wait()

    return sc_gather


def kernel(sentence, table):
    q = sentence.astype(jnp.int32).reshape(128, 128, 25, 8).transpose(2, 0, 3, 1)
    q_flat = q.reshape(B * L)
    out5 = _sc_kernel()(q_flat, table)
    return out5.transpose(2, 4, 0, 1, 3).reshape(B, L, D)


# speedup vs baseline: 3.6171x; 1.7586x over previous
import functools

import jax
import jax.numpy as jnp
from jax import lax
from jax.experimental import pallas as pl
from jax.experimental.pallas import tpu as pltpu
from jax.experimental.pallas import tpu_sc as plsc

V = 1000000
D = 32
B = 16384
L = 200
NC = 2
NS = 16
NW = NC * NS
UNIT = 512
N_UNITS = B * L // UNIT
PER_W = N_UNITS // NW
ROUNDS = PER_W // 2


def _sc_kernel():
    mesh = plsc.VectorSubcoreMesh(core_axis_name="c", subcore_axis_name="s")

    scratch = (
        [pltpu.VMEM((UNIT,), jnp.int32) for _ in range(2)]
        + [pltpu.VMEM((UNIT, D), jnp.float32) for _ in range(2)]
        + [pltpu.VMEM((4, 4, 8, 129), jnp.float32) for _ in range(2)]
        + [pltpu.SemaphoreType.DMA for _ in range(4)]
    )

    @functools.partial(
        pl.kernel,
        mesh=mesh,
        out_type=jax.ShapeDtypeStruct((L, D // 8, B // 128, 8, 128), jnp.float32),
        scratch_types=scratch,
        compiler_params=pltpu.CompilerParams(
            use_tc_tiling_on_sc=False, needs_layout_passes=False),
    )
    def sc_gather(q_hbm, table_hbm, out_hbm, idx0, idx1, r0, r1, t0, t1,
                  gsem0, gsem1, wsem0, wsem1):
        idx_v = (idx0, idx1)
        r_v = (r0, r1)
        t_v = (t0, t1)
        gsem = (gsem0, gsem1)
        wsem = (wsem0, wsem1)

        wid = lax.axis_index("s") * NC + lax.axis_index("c")
        u_base = wid * PER_W
        ramp = lax.iota(jnp.int32, 16)

        def idx_copy(p, u):
            return pltpu.make_async_copy(
                q_hbm.at[pl.ds(u * UNIT, UNIT)], idx_v[p], gsem[p])

        def gather_copy(p):
            return pltpu.make_async_copy(table_hbm.at[idx_v[p]], r_v[p], gsem[p])

        def out_copy(p, u):
            rt = u // 256
            rem = u % 256
            ct = rem // 2
            h = rem % 2
            l0 = rt * 8 + h * 4
            return pltpu.make_async_copy(
                t_v[p].at[:, :, :, pl.ds(0, 128)],
                out_hbm.at[pl.ds(l0, 4), :, ct], wsem[p])

        tr_vec0 = ramp // 8
        dr_vec0 = lax.rem(ramp, 8)
        tr_vec1 = (ramp + 16) // 8
        dr_vec1 = lax.rem(ramp + 16, 8)

        def transpose(p):
            def tl(lrh, carry):
                l_s = jnp.broadcast_to(lrh, (16,))
                base_row = lrh * 128

                @functools.partial(plsc.parallel_loop, 0, 128, unroll=8)
                def _(bc):
                    bc_s = jnp.broadcast_to(bc, (16,))
                    row = base_row + bc
                    v0 = r_v[p][row, pl.ds(0, 16)]
                    plsc.store_scatter(
                        t_v[p], [l_s, tr_vec0, dr_vec0, bc_s], v0)
                    v1 = r_v[p][row, pl.ds(16, 16)]
                    plsc.store_scatter(
                        t_v[p], [l_s, tr_vec1, dr_vec1, bc_s], v1)

                return carry

            lax.fori_loop(0, 4, tl, 0)

        idx_copy(0, u_base).start()
        idx_copy(0, u_base).wait()
        gather_copy(0).start()
        idx_copy(1, u_base + 1).start()
        idx_copy(1, u_base + 1).wait()
        gather_copy(1).start()

        def body(rnd, carry):
            ua = u_base + 2 * rnd
            ub = ua + 1
            gather_copy(0).wait()

            @pl.when(rnd > 0)
            def _():
                out_copy(0, ua - 2).wait()

            transpose(0)
            out_copy(0, ua).start()

            @pl.when(rnd + 1 < ROUNDS)
            def _():
                idx_copy(0, ua + 2).start()
                idx_copy(0, ua + 2).wait()
                gather_copy(0).start()

            gather_copy(1).wait()

            @pl.when(rnd > 0)
            def _():
                out_copy(1, ub - 2).wait()

            transpose(1)
            out_copy(1, ub).start()

            @pl.when(rnd + 1 < ROUNDS)
            def _():
                idx_copy(1, ub + 2).start()
                idx_copy(1, ub + 2).wait()
                gather_copy(1).start()

            return carry

        lax.fori_loop(0, ROUNDS, body, 0)

        last_a = u_base + 2 * (ROUNDS - 1)
        out_copy(0, last_a).wait()
        out_copy(1, last_a + 1).wait()

    return sc_gather


def kernel(sentence, table):
    q = sentence.astype(jnp.int32).reshape(128, 128, 25, 8).transpose(2, 0, 3, 1)
    q_flat = q.reshape(B * L)
    out5 = _sc_kernel()(q_flat, table)
    return out5.transpose(2, 4, 0, 1, 3).reshape(B, L, D)
